# baseline (speedup 1.0000x reference)
import functools

import jax
import jax.numpy as jnp
from jax import lax
from jax.experimental import pallas as pl
from jax.experimental.pallas import tpu as pltpu
from jax.experimental.pallas import tpu_sc as plsc

_B, _C, _H, _W = 8, 64, 128, 128
_K = 512
_N = _H * _W
_RB = 32
_BLK = _RB * _W
_NB = _H // _RB
_STEPS = _B * _NB

_CP = _C + 1
_NC, _NS, _L = 2, 16, 16
_NWORK = _NC * _NS
_WPB = _NWORK // _B
_ROWS_W = _H // _WPB
_SUB = 8
_NSUB = _ROWS_W // _SUB


def _argmin_body(x_ref, cb_ref, kio_ref, idx_ref):
    xb = x_ref[0].reshape(_C, _BLK)
    cb = cb_ref[...]
    kio = kio_ref[...]

    mm = jax.lax.dot(cb, xb, precision=jax.lax.Precision.DEFAULT,
                     preferred_element_type=jnp.float32)
    xsq = jnp.sum(xb * xb, axis=0, keepdims=True)
    e2 = jnp.sum(cb * cb, axis=1, keepdims=True)
    d = (xsq - 2.0 * mm) + e2

    m = jnp.min(d, axis=0, keepdims=True)
    kidf = jnp.min(jnp.where(d == m, kio, float(_K)), axis=0, keepdims=True)
    idx_ref[0] = kidf.astype(jnp.int32).reshape(_RB, _W)


def _tc_argmin(x, codebook):
    kio = jax.lax.broadcasted_iota(jnp.int32, (_K, _BLK), 0).astype(jnp.float32)
    return pl.pallas_call(
        _argmin_body,
        grid=(_STEPS,),
        in_specs=[
            pl.BlockSpec((1, _C, _RB, _W), lambda i: (i // _NB, 0, i % _NB, 0)),
            pl.BlockSpec((_K, _C), lambda i: (0, 0)),
            pl.BlockSpec((_K, _BLK), lambda i: (0, 0)),
        ],
        out_specs=pl.BlockSpec((1, _RB, _W), lambda i: (i // _NB, i % _NB, 0)),
        out_shape=jax.ShapeDtypeStruct((_B, _H, _W), jnp.int32),
    )(x, codebook, kio)


def _sc_body(x_hbm, idx_hbm, cb_hbm, xrec_hbm, part_hbm,
             cb_v, idx_v, xbuf, part_v):
    wid = lax.axis_index("s") * _NC + lax.axis_index("c")
    b = wid // _WPB
    r0 = (wid % _WPB) * _ROWS_W

    pltpu.sync_copy(cb_hbm, cb_v)
    pltpu.sync_copy(idx_hbm.at[b, pl.ds(r0, _ROWS_W)], idx_v)

    _NACC = 8
    accs = tuple(jnp.zeros((_L,), jnp.float32) for _ in range(_NACC))
    for s in range(_NSUB):
        rr = r0 + s * _SUB
        pltpu.sync_copy(x_hbm.at[b, :, pl.ds(rr, _SUB)], xbuf)

        def _row(r, accs):
            def _vec(v, accs):
                i16 = idx_v[s * _SUB + r, pl.ds(v * _L, _L)]
                base = i16 * _CP
                accs = list(accs)
                for c in range(_C):
                    g = plsc.load_gather(cb_v, [base + c])
                    z = xbuf[c, r, pl.ds(v * _L, _L)]
                    dlt = g - z
                    xbuf[c, r, pl.ds(v * _L, _L)] = z + dlt
                    accs[c % _NACC] = accs[c % _NACC] + dlt * dlt
                return tuple(accs)
            return plsc.parallel_loop(0, _W // _L, carry=accs, unroll=4)(_vec)
        accs = lax.fori_loop(0, _SUB, _row, accs)

        pltpu.sync_copy(xbuf, xrec_hbm.at[b, :, pl.ds(rr, _SUB)])

    acc = accs[0]
    for a in accs[1:]:
        acc = acc + a
    part_v[...] = acc
    pltpu.sync_copy(part_v, part_hbm.at[wid])


_sc_quantize = functools.partial(
    pl.kernel,
    mesh=plsc.VectorSubcoreMesh(core_axis_name="c", subcore_axis_name="s"),
    compiler_params=pltpu.CompilerParams(needs_layout_passes=False),
    out_type=[
        jax.ShapeDtypeStruct((_B, _C, _H, _W), jnp.float32),
        jax.ShapeDtypeStruct((_NWORK, _L), jnp.float32),
    ],
    scratch_types=[
        pltpu.VMEM((_K * _CP,), jnp.float32),
        pltpu.VMEM((_ROWS_W, _W), jnp.int32),
        pltpu.VMEM((_C, _SUB, _W), jnp.float32),
        pltpu.VMEM((_L,), jnp.float32),
    ],
)(_sc_body)


def kernel(x, codebook):
    idx = _tc_argmin(x, codebook)
    cb_pad = jnp.pad(codebook, ((0, 0), (0, _CP - _C))).reshape(_K * _CP)
    xrec, part = _sc_quantize(x, idx, cb_pad)
    loss = 1.25 * jnp.sum(part) / (_B * _C * _H * _W)
    return xrec, xrec, idx, loss


# speedup vs baseline: 1.0624x; 1.0624x over previous
import functools

import jax
import jax.numpy as jnp
from jax import lax
from jax.experimental import pallas as pl
from jax.experimental.pallas import tpu as pltpu
from jax.experimental.pallas import tpu_sc as plsc

_B, _C, _H, _W = 8, 64, 128, 128
_K = 512
_N = _H * _W
_RB = 16
_BLK = _RB * _W
_NB = _H // _RB
_STEPS = _B * _NB

_CP = _C + 1
_NC, _NS, _L = 2, 16, 16
_NWORK = _NC * _NS
_WPB = _NWORK // _B
_ROWS_W = _H // _WPB
_SUB = 8
_NSUB = _ROWS_W // _SUB


def _argmin_body(x_ref, cb_ref, kio_ref, idx_ref):
    xb = x_ref[0].reshape(_C, _BLK)
    cb = cb_ref[...]
    kio = kio_ref[...]

    mm = jax.lax.dot(cb, xb, precision=jax.lax.Precision.DEFAULT,
                     preferred_element_type=jnp.float32)
    xsq = jnp.sum(xb * xb, axis=0, keepdims=True)
    e2 = jnp.sum(cb * cb, axis=1, keepdims=True)
    d = (xsq - 2.0 * mm) + e2

    m = jnp.min(d, axis=0, keepdims=True)
    kidf = jnp.min(jnp.where(d == m, kio, float(_K)), axis=0, keepdims=True)
    idx_ref[0] = kidf.astype(jnp.int32).reshape(_RB, _W)


def _tc_argmin(x, codebook):
    kio = jax.lax.broadcasted_iota(jnp.int32, (_K, _BLK), 0).astype(jnp.float32)
    return pl.pallas_call(
        _argmin_body,
        grid=(_STEPS,),
        in_specs=[
            pl.BlockSpec((1, _C, _RB, _W), lambda i: (i // _NB, 0, i % _NB, 0)),
            pl.BlockSpec((_K, _C), lambda i: (0, 0)),
            pl.BlockSpec((_K, _BLK), lambda i: (0, 0)),
        ],
        out_specs=pl.BlockSpec((1, _RB, _W), lambda i: (i // _NB, i % _NB, 0)),
        out_shape=jax.ShapeDtypeStruct((_B, _H, _W), jnp.int32),
    )(x, codebook, kio)


def _sc_body(x_hbm, idx_hbm, cb_hbm, xrec_hbm, part_hbm,
             cb_v, idx_v, xbuf, part_v):
    wid = lax.axis_index("s") * _NC + lax.axis_index("c")
    b = wid // _WPB
    r0 = (wid % _WPB) * _ROWS_W

    pltpu.sync_copy(cb_hbm, cb_v)
    pltpu.sync_copy(idx_hbm.at[b, pl.ds(r0, _ROWS_W)], idx_v)

    _NACC = 8
    accs = tuple(jnp.zeros((_L,), jnp.float32) for _ in range(_NACC))
    for s in range(_NSUB):
        rr = r0 + s * _SUB
        pltpu.sync_copy(x_hbm.at[b, :, pl.ds(rr, _SUB)], xbuf)

        _VPR = _W // _L

        def _vec(i, accs):
            r = i // _VPR
            v = i % _VPR
            i16 = idx_v[s * _SUB + r, pl.ds(v * _L, _L)]
            base = i16 * _CP
            accs = list(accs)
            for c in range(_C):
                g = plsc.load_gather(cb_v, [base + c])
                z = xbuf[c, r, pl.ds(v * _L, _L)]
                dlt = g - z
                xbuf[c, r, pl.ds(v * _L, _L)] = z + dlt
                accs[c % _NACC] = accs[c % _NACC] + dlt * dlt
            return tuple(accs)
        accs = plsc.parallel_loop(0, _SUB * _VPR, carry=accs, unroll=4)(_vec)

        pltpu.sync_copy(xbuf, xrec_hbm.at[b, :, pl.ds(rr, _SUB)])

    acc = accs[0]
    for a in accs[1:]:
        acc = acc + a
    part_v[...] = acc
    pltpu.sync_copy(part_v, part_hbm.at[wid])


_sc_quantize = functools.partial(
    pl.kernel,
    mesh=plsc.VectorSubcoreMesh(core_axis_name="c", subcore_axis_name="s"),
    compiler_params=pltpu.CompilerParams(needs_layout_passes=False),
    out_type=[
        jax.ShapeDtypeStruct((_B, _C, _H, _W), jnp.float32),
        jax.ShapeDtypeStruct((_NWORK, _L), jnp.float32),
    ],
    scratch_types=[
        pltpu.VMEM((_K * _CP,), jnp.float32),
        pltpu.VMEM((_ROWS_W, _W), jnp.int32),
        pltpu.VMEM((_C, _SUB, _W), jnp.float32),
        pltpu.VMEM((_L,), jnp.float32),
    ],
)(_sc_body)


def kernel(x, codebook):
    idx = _tc_argmin(x, codebook)
    cb_pad = jnp.pad(codebook, ((0, 0), (0, _CP - _C))).reshape(_K * _CP)
    xrec, part = _sc_quantize(x, idx, cb_pad)
    loss = 1.25 * jnp.sum(part) / (_B * _C * _H * _W)
    return xrec, xrec, idx, loss


# speedup vs baseline: 1.1130x; 1.0476x over previous
import functools

import jax
import jax.numpy as jnp
from jax import lax
from jax.experimental import pallas as pl
from jax.experimental.pallas import tpu as pltpu
from jax.experimental.pallas import tpu_sc as plsc

_B, _C, _H, _W = 8, 64, 128, 128
_K = 512
_N = _H * _W
_RB = 16
_BLK = _RB * _W
_NB = _H // _RB
_STEPS = _B * _NB

_CP = _C + 1
_NC, _NS, _L = 2, 16, 16
_NWORK = _NC * _NS
_WPB = _NWORK // _B
_ROWS_W = _H // _WPB
_SUB = 4
_NSUB = _ROWS_W // _SUB


def _argmin_body(x_ref, cb_ref, kio_ref, idx_ref):
    xb = x_ref[0].reshape(_C, _BLK)
    cb = cb_ref[...]
    kio = kio_ref[...]

    mm = jax.lax.dot(cb, xb, precision=jax.lax.Precision.DEFAULT,
                     preferred_element_type=jnp.float32)
    xsq = jnp.sum(xb * xb, axis=0, keepdims=True)
    e2 = jnp.sum(cb * cb, axis=1, keepdims=True)
    d = (xsq - 2.0 * mm) + e2

    m = jnp.min(d, axis=0, keepdims=True)
    kidf = jnp.min(jnp.where(d == m, kio, float(_K)), axis=0, keepdims=True)
    idx_ref[0] = kidf.astype(jnp.int32).reshape(_RB, _W)


def _tc_argmin(x, codebook):
    kio = jax.lax.broadcasted_iota(jnp.int32, (_K, _BLK), 0).astype(jnp.float32)
    return pl.pallas_call(
        _argmin_body,
        grid=(_STEPS,),
        in_specs=[
            pl.BlockSpec((1, _C, _RB, _W), lambda i: (i // _NB, 0, i % _NB, 0)),
            pl.BlockSpec((_K, _C), lambda i: (0, 0)),
            pl.BlockSpec((_K, _BLK), lambda i: (0, 0)),
        ],
        out_specs=pl.BlockSpec((1, _RB, _W), lambda i: (i // _NB, i % _NB, 0)),
        out_shape=jax.ShapeDtypeStruct((_B, _H, _W), jnp.int32),
    )(x, codebook, kio)


def _sc_body(x_hbm, idx_hbm, cb_hbm, xrec_hbm, part_hbm,
             cb_v, idx_v, xbuf0, xbuf1, part_v,
             sin0, sin1, sout0, sout1):
    wid = lax.axis_index("s") * _NC + lax.axis_index("c")
    b = wid // _WPB
    r0 = (wid % _WPB) * _ROWS_W

    pltpu.sync_copy(cb_hbm, cb_v)
    pltpu.sync_copy(idx_hbm.at[b, pl.ds(r0, _ROWS_W)], idx_v)

    bufs = (xbuf0, xbuf1)
    sins = (sin0, sin1)
    souts = (sout0, sout1)

    def _in_copy(s):
        return pltpu.make_async_copy(
            x_hbm.at[b, :, pl.ds(r0 + s * _SUB, _SUB)], bufs[s % 2], sins[s % 2])

    def _out_copy(s):
        return pltpu.make_async_copy(
            bufs[s % 2], xrec_hbm.at[b, :, pl.ds(r0 + s * _SUB, _SUB)],
            souts[s % 2])

    _NACC = 8
    _VPR = _W // _L
    accs = tuple(jnp.zeros((_L,), jnp.float32) for _ in range(_NACC))

    _in_copy(0).start()
    for s in range(_NSUB):
        xbuf = bufs[s % 2]
        if s + 1 < _NSUB:
            if s >= 1:
                _out_copy(s - 1).wait()
            _in_copy(s + 1).start()
        _in_copy(s).wait()

        def _vec(i, accs, s=s, xbuf=xbuf):
            r = i // _VPR
            v = i % _VPR
            i16 = idx_v[s * _SUB + r, pl.ds(v * _L, _L)]
            base = i16 * _CP
            accs = list(accs)
            for c in range(_C):
                g = plsc.load_gather(cb_v, [base + c])
                z = xbuf[c, r, pl.ds(v * _L, _L)]
                dlt = g - z
                xbuf[c, r, pl.ds(v * _L, _L)] = z + dlt
                accs[c % _NACC] = accs[c % _NACC] + dlt * dlt
            return tuple(accs)
        accs = plsc.parallel_loop(0, _SUB * _VPR, carry=accs, unroll=2)(_vec)

        _out_copy(s).start()

    _out_copy(_NSUB - 2).wait()
    _out_copy(_NSUB - 1).wait()

    acc = accs[0]
    for a in accs[1:]:
        acc = acc + a
    part_v[...] = acc
    pltpu.sync_copy(part_v, part_hbm.at[wid])


_sc_quantize = functools.partial(
    pl.kernel,
    mesh=plsc.VectorSubcoreMesh(core_axis_name="c", subcore_axis_name="s"),
    compiler_params=pltpu.CompilerParams(needs_layout_passes=False),
    out_type=[
        jax.ShapeDtypeStruct((_B, _C, _H, _W), jnp.float32),
        jax.ShapeDtypeStruct((_NWORK, _L), jnp.float32),
    ],
    scratch_types=[
        pltpu.VMEM((_K * _CP,), jnp.float32),
        pltpu.VMEM((_ROWS_W, _W), jnp.int32),
        pltpu.VMEM((_C, _SUB, _W), jnp.float32),
        pltpu.VMEM((_C, _SUB, _W), jnp.float32),
        pltpu.VMEM((_L,), jnp.float32),
        pltpu.SemaphoreType.DMA,
        pltpu.SemaphoreType.DMA,
        pltpu.SemaphoreType.DMA,
        pltpu.SemaphoreType.DMA,
    ],
)(_sc_body)


def kernel(x, codebook):
    idx = _tc_argmin(x, codebook)
    cb_pad = jnp.pad(codebook, ((0, 0), (0, _CP - _C))).reshape(_K * _CP)
    xrec, part = _sc_quantize(x, idx, cb_pad)
    loss = 1.25 * jnp.sum(part) / (_B * _C * _H * _W)
    return xrec, xrec, idx, loss
